# baseline (speedup 1.0000x reference)
import functools
import jax
import jax.numpy as jnp
from jax import lax
from jax.experimental import pallas as pl
from jax.experimental.pallas import tpu as pltpu
from jax.experimental.pallas import tpu_sc as plsc

D_MODEL = 64
SCALE = 8.0
LANES = 16
CHUNK = 128
NBUF = 4
UNIT = 8
TBLK = 1024


def _pack_table(lut_t):
    n = lut_t.shape[1]
    grid = (n + TBLK - 1) // TBLK

    def body(in_ref, out_ref):
        x = in_ref[...]
        x3 = x.reshape(D_MODEL, TBLK // 2, 2)
        even = x3[:, :, 0].T
        odd = x3[:, :, 1].T
        out_ref[...] = jnp.concatenate([even, odd], axis=1)

    return pl.pallas_call(
        body,
        grid=(grid,),
        in_specs=[pl.BlockSpec((D_MODEL, TBLK), lambda g: (0, g))],
        out_specs=pl.BlockSpec((TBLK // 2, 2 * D_MODEL), lambda g: (g, 0)),
        out_shape=jax.ShapeDtypeStruct((n // 2, 2 * D_MODEL), jnp.float32),
    )(lut_t)


def _to_native(inter4):
    seq, nbg = inter4.shape[0], inter4.shape[1]

    def body(in_ref, out_ref):
        x = in_ref[0, 0]
        lo = x[:, 0:D_MODEL].T * SCALE
        hi = x[:, D_MODEL:].T * SCALE
        y = jnp.concatenate([lo, hi], axis=1)
        out_ref[0, :, 0] = y.reshape(D_MODEL // 8, 8, CHUNK)

    return pl.pallas_call(
        body,
        grid=(seq, nbg),
        in_specs=[
            pl.BlockSpec((1, 1, D_MODEL, 2 * D_MODEL), lambda s, g: (s, g, 0, 0))
        ],
        out_specs=pl.BlockSpec(
            (1, D_MODEL // 8, 1, 8, CHUNK), lambda s, g: (s, 0, g, 0, 0)
        ),
        out_shape=jax.ShapeDtypeStruct(
            (seq, D_MODEL // 8, nbg, 8, CHUNK), jnp.float32
        ),
    )(inter4)


def kernel(lut, x):
    b_total, seq = x.shape
    xt = x.T.astype(jnp.int32).reshape(seq, b_total // 1024, UNIT, CHUNK)
    table = _pack_table(lut.T).reshape(-1, D_MODEL)

    info = plsc.get_sparse_core_info()
    num_workers = info.num_cores * info.num_subcores
    num_cores = info.num_cores
    chunks_per_s = b_total // CHUNK
    n_chunks = (b_total * seq) // (CHUNK * num_workers)

    mesh = plsc.VectorSubcoreMesh(core_axis_name="c", subcore_axis_name="s")

    @functools.partial(
        pl.kernel,
        mesh=mesh,
        out_type=jax.ShapeDtypeStruct(
            (seq, b_total // CHUNK, CHUNK, D_MODEL), jnp.float32
        ),
        scratch_types=[
            pltpu.VMEM((UNIT, CHUNK), jnp.int32),
            [pltpu.VMEM((CHUNK,), jnp.int32) for _ in range(NBUF)],
            [pltpu.VMEM((CHUNK, D_MODEL), jnp.float32) for _ in range(NBUF)],
            [pltpu.SemaphoreType.DMA for _ in range(NBUF)],
            [pltpu.SemaphoreType.DMA for _ in range(NBUF)],
        ],
        compiler_params=pltpu.CompilerParams(
            needs_layout_passes=False, use_tc_tiling_on_sc=False
        ),
    )
    def gather_core(lut_hbm, x_hbm, out_hbm, islot, ichunks, gbufs, gsems, wsems):
        wid = lax.axis_index("s") * num_cores + lax.axis_index("c")
        k0 = wid * n_chunks

        def out_coords(j):
            kg = k0 + j
            return kg >> 5, kg & (chunks_per_s - 1)

        iota2 = jax.lax.iota(jnp.int32, LANES) * 2

        def prep_idx(j, b):
            kg = k0 + j
            cc = kg & (UNIT - 1)

            @pl.when(cc == 0)
            def _():
                pltpu.sync_copy(x_hbm.at[kg >> 5, (kg >> 3) & 3], islot)

            for l in range(CHUNK // LANES):
                v = islot[cc, pl.ds(l * LANES, LANES)]
                off = l * 2 * LANES if l < 4 else (l - 4) * 2 * LANES + 1
                plsc.store_scatter(ichunks[b], [iota2 + off], v)

        def gather_start(j, b):
            pltpu.async_copy(lut_hbm.at[ichunks[b]], gbufs[b], gsems[b])

        def gather_wait(j, b):
            pltpu.make_async_copy(lut_hbm.at[ichunks[b]], gbufs[b], gsems[b]).wait()

        def write_start(j, b):
            s, bg = out_coords(j)
            pltpu.async_copy(gbufs[b], out_hbm.at[s, bg], wsems[b])

        def write_wait(j, b):
            s, bg = out_coords(j)
            pltpu.make_async_copy(gbufs[b], out_hbm.at[s, bg], wsems[b]).wait()

        prep_idx(0, 0)
        gather_start(0, 0)
        prep_idx(1, 1)
        gather_start(1, 1)
        for j in range(2):
            prep_idx(j + 2, (j + 2) % NBUF)
            gather_start(j + 2, (j + 2) % NBUF)
            gather_wait(j, j % NBUF)
            write_start(j, j % NBUF)

        def steady(ci, carry):
            for u in range(NBUF):
                j = 2 + ci * NBUF + u
                b = (2 + u) % NBUF
                write_wait(j - 2, u % NBUF)
                prep_idx(j + 2, u % NBUF)
                gather_start(j + 2, u % NBUF)
                gather_wait(j, b)
                write_start(j, b)
            return carry

        lax.fori_loop(0, (n_chunks - NBUF) // NBUF, steady, 0)

        for j in range(n_chunks - 2, n_chunks):
            gather_wait(j, j % NBUF)
            write_start(j, j % NBUF)
        for j in range(n_chunks - NBUF, n_chunks):
            write_wait(j, j % NBUF)

    inter = gather_core(table, xt)
    out = _to_native(inter.reshape(seq, b_total // CHUNK, D_MODEL, 2 * D_MODEL))
    return out.transpose(2, 4, 0, 1, 3).reshape(b_total, seq, D_MODEL)


# speedup vs baseline: 3.7998x; 3.7998x over previous
import functools
import jax
import jax.numpy as jnp
from jax import lax
from jax.experimental import pallas as pl
from jax.experimental.pallas import tpu as pltpu
from jax.experimental.pallas import tpu_sc as plsc

D_MODEL = 64
SCALE = 8.0
LANES = 16
CHUNK = 128
NBUF = 4
UNIT = 8
TBLK = 1024
_TDOT = (((0,), (0,)), ((), ()))


def _pack_table(lut_t):
    n = lut_t.shape[1]
    grid = (n + TBLK - 1) // TBLK

    def body(in_ref, out_ref):
        x = in_ref[...]
        eye = jnp.eye(D_MODEL, dtype=jnp.float32)
        xt = lax.dot_general(
            x, eye, _TDOT, preferred_element_type=jnp.float32
        )
        out_ref[...] = jnp.concatenate(
            [xt, jnp.zeros((TBLK, D_MODEL), jnp.float32)], axis=1
        )

    return pl.pallas_call(
        body,
        grid=(grid,),
        in_specs=[pl.BlockSpec((D_MODEL, TBLK), lambda g: (0, g))],
        out_specs=pl.BlockSpec((TBLK, 2 * D_MODEL), lambda g: (g, 0)),
        out_shape=jax.ShapeDtypeStruct((n, 2 * D_MODEL), jnp.float32),
    )(lut_t)


def _to_native(inter):
    seq, nbg = inter.shape[0], inter.shape[1]

    def body(in_ref, out_ref):
        x = in_ref[0, 0]
        eye = jnp.eye(CHUNK, dtype=jnp.float32)
        y = lax.dot_general(
            x, eye, _TDOT, preferred_element_type=jnp.float32
        )
        out_ref[0, :, 0] = (y[0:D_MODEL, :] * SCALE).reshape(
            D_MODEL // 8, 8, CHUNK
        )

    return pl.pallas_call(
        body,
        grid=(seq, nbg),
        in_specs=[
            pl.BlockSpec((1, 1, CHUNK, CHUNK), lambda s, g: (s, g, 0, 0))
        ],
        out_specs=pl.BlockSpec(
            (1, D_MODEL // 8, 1, 8, CHUNK), lambda s, g: (s, 0, g, 0, 0)
        ),
        out_shape=jax.ShapeDtypeStruct(
            (seq, D_MODEL // 8, nbg, 8, CHUNK), jnp.float32
        ),
    )(inter)


def kernel(lut, x):
    b_total, seq = x.shape
    xt = x.T.astype(jnp.int32).reshape(seq, b_total // 1024, UNIT, CHUNK)
    table = _pack_table(lut.T)

    info = plsc.get_sparse_core_info()
    num_workers = info.num_cores * info.num_subcores
    num_cores = info.num_cores
    chunks_per_s = b_total // CHUNK
    n_chunks = (b_total * seq) // (CHUNK * num_workers)

    mesh = plsc.VectorSubcoreMesh(core_axis_name="c", subcore_axis_name="s")

    @functools.partial(
        pl.kernel,
        mesh=mesh,
        out_type=jax.ShapeDtypeStruct(
            (seq, b_total // CHUNK, CHUNK, CHUNK), jnp.float32
        ),
        scratch_types=[
            pltpu.VMEM((UNIT, CHUNK), jnp.int32),
            [pltpu.VMEM((CHUNK,), jnp.int32) for _ in range(NBUF)],
            [pltpu.VMEM((CHUNK, CHUNK), jnp.float32) for _ in range(NBUF)],
            [pltpu.SemaphoreType.DMA for _ in range(NBUF)],
            [pltpu.SemaphoreType.DMA for _ in range(NBUF)],
        ],
        compiler_params=pltpu.CompilerParams(
            needs_layout_passes=False, use_tc_tiling_on_sc=False
        ),
    )
    def gather_core(lut_hbm, x_hbm, out_hbm, islot, ichunks, gbufs, gsems, wsems):
        wid = lax.axis_index("s") * num_cores + lax.axis_index("c")
        k0 = wid * n_chunks

        def out_coords(j):
            kg = k0 + j
            return kg >> 5, kg & (chunks_per_s - 1)

        def prep_idx(j, b):
            kg = k0 + j
            cc = kg & (UNIT - 1)

            @pl.when(cc == 0)
            def _():
                pltpu.sync_copy(x_hbm.at[kg >> 5, (kg >> 3) & 3], islot)

            for l in range(CHUNK // LANES):
                sl = pl.ds(l * LANES, LANES)
                ichunks[b][sl] = islot[cc, sl]

        def gather_start(j, b):
            pltpu.async_copy(lut_hbm.at[ichunks[b]], gbufs[b], gsems[b])

        def gather_wait(j, b):
            pltpu.make_async_copy(lut_hbm.at[ichunks[b]], gbufs[b], gsems[b]).wait()

        def write_start(j, b):
            s, bg = out_coords(j)
            pltpu.async_copy(gbufs[b], out_hbm.at[s, bg], wsems[b])

        def write_wait(j, b):
            s, bg = out_coords(j)
            pltpu.make_async_copy(gbufs[b], out_hbm.at[s, bg], wsems[b]).wait()

        prep_idx(0, 0)
        gather_start(0, 0)
        prep_idx(1, 1)
        gather_start(1, 1)
        for j in range(2):
            prep_idx(j + 2, (j + 2) % NBUF)
            gather_start(j + 2, (j + 2) % NBUF)
            gather_wait(j, j % NBUF)
            write_start(j, j % NBUF)

        def steady(ci, carry):
            for u in range(NBUF):
                j = 2 + ci * NBUF + u
                b = (2 + u) % NBUF
                write_wait(j - 2, u % NBUF)
                prep_idx(j + 2, u % NBUF)
                gather_start(j + 2, u % NBUF)
                gather_wait(j, b)
                write_start(j, b)
            return carry

        lax.fori_loop(0, (n_chunks - NBUF) // NBUF, steady, 0)

        for j in range(n_chunks - 2, n_chunks):
            gather_wait(j, j % NBUF)
            write_start(j, j % NBUF)
        for j in range(n_chunks - NBUF, n_chunks):
            write_wait(j, j % NBUF)

    inter = gather_core(table, xt)
    out = _to_native(inter)
    return out.transpose(2, 4, 0, 1, 3).reshape(b_total, seq, D_MODEL)


# speedup vs baseline: 14.3457x; 3.7754x over previous
import functools
import jax
import jax.numpy as jnp
from jax import lax
from jax.experimental import pallas as pl
from jax.experimental.pallas import tpu as pltpu
from jax.experimental.pallas import tpu_sc as plsc

D_MODEL = 64
SCALE = 8.0
LANES = 16
CHUNK = 320
NBUF = 4


def kernel(lut, x):
    batch_shape = x.shape
    xf = x.reshape(-1).astype(jnp.int32)
    total = xf.shape[0]

    info = plsc.get_sparse_core_info()
    num_workers = info.num_cores * info.num_subcores
    per_worker = total // num_workers
    n_chunks = per_worker // CHUNK
    num_cores = info.num_cores

    mesh = plsc.VectorSubcoreMesh(core_axis_name="c", subcore_axis_name="s")

    @functools.partial(
        pl.kernel,
        mesh=mesh,
        out_type=jax.ShapeDtypeStruct((total, D_MODEL), jnp.float32),
        scratch_types=[
            pltpu.VMEM((per_worker,), jnp.int32),
            [pltpu.VMEM((CHUNK, D_MODEL), jnp.float32) for _ in range(NBUF)],
            [pltpu.SemaphoreType.DMA for _ in range(NBUF)],
            [pltpu.SemaphoreType.DMA for _ in range(NBUF)],
        ],
        compiler_params=pltpu.CompilerParams(use_tc_tiling_on_sc=False),
    )
    def gather_scale(lut_hbm, x_hbm, out_hbm, idx_v, bufs, gsems, wsems):
        wid = lax.axis_index("s") * num_cores + lax.axis_index("c")
        base = wid * per_worker
        pltpu.sync_copy(x_hbm.at[pl.ds(base, per_worker)], idx_v)

        def gather_start(j, b):
            pltpu.async_copy(
                lut_hbm.at[idx_v.at[pl.ds(j * CHUNK, CHUNK)]], bufs[b], gsems[b]
            )

        def gather_wait(j, b):
            pltpu.make_async_copy(
                lut_hbm.at[idx_v.at[pl.ds(j * CHUNK, CHUNK)]], bufs[b], gsems[b]
            ).wait()

        def write_start(j, b):
            pltpu.async_copy(
                bufs[b], out_hbm.at[pl.ds(base + j * CHUNK, CHUNK)], wsems[b]
            )

        def write_wait(j, b):
            pltpu.make_async_copy(
                bufs[b], out_hbm.at[pl.ds(base + j * CHUNK, CHUNK)], wsems[b]
            ).wait()

        def scale(b):
            buf = bufs[b]

            def row_body(r2, carry):
                r = r2 * 2
                for u in range(2):
                    for q in range(D_MODEL // LANES):
                        buf[r + u, pl.ds(q * LANES, LANES)] = (
                            buf[r + u, pl.ds(q * LANES, LANES)] * SCALE
                        )
                return carry

            lax.fori_loop(0, CHUNK // 2, row_body, 0)

        gather_start(0, 0)
        for j in range(NBUF - 1):
            gather_start(j + 1, j + 1)
            gather_wait(j, j)
            scale(j)
            write_start(j, j)

        def steady(c, carry):
            for b in range(NBUF):
                j = (NBUF - 1) + c * NBUF + b
                bf = (NBUF - 1 + b) % NBUF
                write_wait(j - (NBUF - 1), b)
                gather_start(j + 1, b)
                gather_wait(j, bf)
                scale(bf)
                write_start(j, bf)
            return carry

        lax.fori_loop(0, (n_chunks - NBUF) // NBUF, steady, 0)

        jl = n_chunks - 1
        bl = jl % NBUF
        gather_wait(jl, bl)
        scale(bl)
        write_start(jl, bl)
        for j in range(n_chunks - NBUF, n_chunks):
            write_wait(j, j % NBUF)

    out = gather_scale(lut, xf)
    return out.reshape(batch_shape + (D_MODEL,))


# speedup vs baseline: 20.8172x; 1.4511x over previous
import functools
import jax
import jax.numpy as jnp
from jax import lax
from jax.experimental import pallas as pl
from jax.experimental.pallas import tpu as pltpu
from jax.experimental.pallas import tpu_sc as plsc

D_MODEL = 64
SCALE = 8.0
LANES = 16
CHUNK = 128
NBUF = 4
UNIT = 8
TBLK = 8192


def _pack_table(lut_t):
    n = lut_t.shape[1]
    grid = (n + TBLK - 1) // TBLK

    def body(in_ref, out_ref):
        x = in_ref[...]
        eye = jnp.eye(D_MODEL, dtype=jnp.float32)
        xt = lax.dot_general(
            x, eye, (((0,), (0,)), ((), ())), preferred_element_type=jnp.float32
        )
        out_ref[...] = jnp.concatenate(
            [xt, jnp.zeros((TBLK, D_MODEL), jnp.float32)], axis=1
        )

    return pl.pallas_call(
        body,
        grid=(grid,),
        in_specs=[pl.BlockSpec((D_MODEL, TBLK), lambda g: (0, g))],
        out_specs=pl.BlockSpec((TBLK, 2 * D_MODEL), lambda g: (g, 0)),
        out_shape=jax.ShapeDtypeStruct((n, 2 * D_MODEL), jnp.float32),
    )(lut_t)


def _to_native(inter):
    seq, nbg = inter.shape[0], inter.shape[1]

    def body(in_ref, out_ref):
        x = in_ref[0]
        eye = jnp.eye(CHUNK, dtype=jnp.float32)
        y = lax.dot_general(
            x, eye, (((1,), (0,)), ((), ())), preferred_element_type=jnp.float32
        )
        y = y[:, 0:D_MODEL, :] * SCALE
        y = y.reshape(nbg, D_MODEL // 8, 8, CHUNK).transpose(1, 0, 2, 3)
        out_ref[0] = y

    return pl.pallas_call(
        body,
        grid=(seq,),
        in_specs=[pl.BlockSpec((1, nbg, CHUNK, CHUNK), lambda s: (s, 0, 0, 0))],
        out_specs=pl.BlockSpec(
            (1, D_MODEL // 8, nbg, 8, CHUNK), lambda s: (s, 0, 0, 0, 0)
        ),
        out_shape=jax.ShapeDtypeStruct(
            (seq, D_MODEL // 8, nbg, 8, CHUNK), jnp.float32
        ),
    )(inter)


def kernel(lut, x):
    b_total, seq = x.shape
    xt = x.T.astype(jnp.int32).reshape(seq, b_total // 1024, UNIT, CHUNK)
    table = _pack_table(lut.T)

    info = plsc.get_sparse_core_info()
    num_workers = info.num_cores * info.num_subcores
    num_cores = info.num_cores
    chunks_per_s = b_total // CHUNK
    n_chunks = (b_total * seq) // (CHUNK * num_workers)

    mesh = plsc.VectorSubcoreMesh(core_axis_name="c", subcore_axis_name="s")

    @functools.partial(
        pl.kernel,
        mesh=mesh,
        out_type=jax.ShapeDtypeStruct(
            (seq, b_total // CHUNK, CHUNK, CHUNK), jnp.float32
        ),
        scratch_types=[
            pltpu.VMEM((UNIT, CHUNK), jnp.int32),
            [pltpu.VMEM((CHUNK,), jnp.int32) for _ in range(NBUF)],
            [pltpu.VMEM((CHUNK, CHUNK), jnp.float32) for _ in range(NBUF)],
            [pltpu.SemaphoreType.DMA for _ in range(NBUF)],
            [pltpu.SemaphoreType.DMA for _ in range(NBUF)],
        ],
        compiler_params=pltpu.CompilerParams(
            needs_layout_passes=False, use_tc_tiling_on_sc=False
        ),
    )
    def gather_core(lut_hbm, x_hbm, out_hbm, islot, ichunks, gbufs, gsems, wsems):
        wid = lax.axis_index("s") * num_cores + lax.axis_index("c")
        k0 = wid * n_chunks

        def out_coords(j):
            kg = k0 + j
            return kg >> 5, kg & (chunks_per_s - 1)

        def prep_idx(j, b):
            kg = k0 + j
            cc = kg & (UNIT - 1)

            @pl.when(cc == 0)
            def _():
                pltpu.sync_copy(x_hbm.at[kg >> 5, (kg >> 3) & 3], islot)

            for l in range(CHUNK // LANES):
                sl = pl.ds(l * LANES, LANES)
                ichunks[b][sl] = islot[cc, sl]

        def gather_start(j, b):
            pltpu.async_copy(lut_hbm.at[ichunks[b]], gbufs[b], gsems[b])

        def gather_wait(j, b):
            pltpu.make_async_copy(lut_hbm.at[ichunks[b]], gbufs[b], gsems[b]).wait()

        def write_start(j, b):
            s, bg = out_coords(j)
            pltpu.async_copy(gbufs[b], out_hbm.at[s, bg], wsems[b])

        def write_wait(j, b):
            s, bg = out_coords(j)
            pltpu.make_async_copy(gbufs[b], out_hbm.at[s, bg], wsems[b]).wait()

        prep_idx(0, 0)
        gather_start(0, 0)
        prep_idx(1, 1)
        gather_start(1, 1)
        for j in range(2):
            prep_idx(j + 2, (j + 2) % NBUF)
            gather_start(j + 2, (j + 2) % NBUF)
            gather_wait(j, j % NBUF)
            write_start(j, j % NBUF)

        def steady(ci, carry):
            for u in range(NBUF):
                j = 2 + ci * NBUF + u
                b = (2 + u) % NBUF
                write_wait(j - 2, u % NBUF)
                prep_idx(j + 2, u % NBUF)
                gather_start(j + 2, u % NBUF)
                gather_wait(j, b)
                write_start(j, b)
            return carry

        lax.fori_loop(0, (n_chunks - NBUF) // NBUF, steady, 0)

        for j in range(n_chunks - 2, n_chunks):
            gather_wait(j, j % NBUF)
            write_start(j, j % NBUF)
        for j in range(n_chunks - NBUF, n_chunks):
            write_wait(j, j % NBUF)

    inter = gather_core(table, xt)
    out = _to_native(inter)
    return out.transpose(2, 4, 0, 1, 3).reshape(b_total, seq, D_MODEL)


# speedup vs baseline: 23.2793x; 1.1183x over previous
import functools
import jax
import jax.numpy as jnp
from jax import lax
from jax.experimental import pallas as pl
from jax.experimental.pallas import tpu as pltpu
from jax.experimental.pallas import tpu_sc as plsc

D_MODEL = 64
SCALE = 8.0
LANES = 16
CHUNK = 128
NBUF = 4
UNIT = 8
TBLK = 16384
SB = 2


def _pack_table(lut_t):
    n = lut_t.shape[1]
    grid = (n + TBLK - 1) // TBLK

    def body(in_ref, out_ref):
        x = in_ref[...]
        eye = jnp.eye(D_MODEL, dtype=jnp.float32)
        xt = lax.dot_general(
            x, eye, (((0,), (0,)), ((), ())), preferred_element_type=jnp.float32
        )
        out_ref[...] = jnp.concatenate(
            [xt, jnp.zeros((TBLK, D_MODEL), jnp.float32)], axis=1
        )

    return pl.pallas_call(
        body,
        grid=(grid,),
        in_specs=[pl.BlockSpec((D_MODEL, TBLK), lambda g: (0, g))],
        out_specs=pl.BlockSpec((TBLK, 2 * D_MODEL), lambda g: (g, 0)),
        out_shape=jax.ShapeDtypeStruct((n, 2 * D_MODEL), jnp.float32),
    )(lut_t)


def _to_native(inter):
    seq, nbg = inter.shape[0], inter.shape[1]

    def body(in_ref, out_ref):
        x = in_ref[...]
        x = x.reshape(SB * nbg, CHUNK, CHUNK)
        eye = jnp.eye(CHUNK, dtype=jnp.float32)
        y = lax.dot_general(
            x, eye, (((1,), (0,)), ((), ())), preferred_element_type=jnp.float32
        )
        y = y[:, 0:D_MODEL, :] * SCALE
        y = y.reshape(SB, nbg, D_MODEL // 8, 8, CHUNK).transpose(0, 2, 1, 3, 4)
        out_ref[...] = y

    return pl.pallas_call(
        body,
        grid=(seq // SB,),
        in_specs=[pl.BlockSpec((SB, nbg, CHUNK, CHUNK), lambda s: (s, 0, 0, 0))],
        out_specs=pl.BlockSpec(
            (SB, D_MODEL // 8, nbg, 8, CHUNK), lambda s: (s, 0, 0, 0, 0)
        ),
        out_shape=jax.ShapeDtypeStruct(
            (seq, D_MODEL // 8, nbg, 8, CHUNK), jnp.float32
        ),
    )(inter)


def kernel(lut, x):
    b_total, seq = x.shape
    xt = x.T.astype(jnp.int32).reshape(seq, b_total // 1024, UNIT, CHUNK)
    table = _pack_table(lut.T)

    info = plsc.get_sparse_core_info()
    num_workers = info.num_cores * info.num_subcores
    num_cores = info.num_cores
    chunks_per_s = b_total // CHUNK
    n_chunks = (b_total * seq) // (CHUNK * num_workers)

    mesh = plsc.VectorSubcoreMesh(core_axis_name="c", subcore_axis_name="s")

    @functools.partial(
        pl.kernel,
        mesh=mesh,
        out_type=jax.ShapeDtypeStruct(
            (seq, b_total // CHUNK, CHUNK, CHUNK), jnp.float32
        ),
        scratch_types=[
            pltpu.VMEM((UNIT, CHUNK), jnp.int32),
            [pltpu.VMEM((CHUNK,), jnp.int32) for _ in range(NBUF)],
            [pltpu.VMEM((CHUNK, CHUNK), jnp.float32) for _ in range(NBUF)],
            [pltpu.SemaphoreType.DMA for _ in range(NBUF)],
            [pltpu.SemaphoreType.DMA for _ in range(NBUF)],
        ],
        compiler_params=pltpu.CompilerParams(
            needs_layout_passes=False, use_tc_tiling_on_sc=False
        ),
    )
    def gather_core(lut_hbm, x_hbm, out_hbm, islot, ichunks, gbufs, gsems, wsems):
        wid = lax.axis_index("s") * num_cores + lax.axis_index("c")
        k0 = wid * n_chunks

        def out_coords(j):
            kg = k0 + j
            return kg >> 5, kg & (chunks_per_s - 1)

        def prep_idx(j, b):
            kg = k0 + j
            cc = kg & (UNIT - 1)

            @pl.when(cc == 0)
            def _():
                pltpu.sync_copy(x_hbm.at[kg >> 5, (kg >> 3) & 3], islot)

            for l in range(CHUNK // LANES):
                sl = pl.ds(l * LANES, LANES)
                ichunks[b][sl] = islot[cc, sl]

        def gather_start(j, b):
            pltpu.async_copy(lut_hbm.at[ichunks[b]], gbufs[b], gsems[b])

        def gather_wait(j, b):
            pltpu.make_async_copy(lut_hbm.at[ichunks[b]], gbufs[b], gsems[b]).wait()

        def write_start(j, b):
            s, bg = out_coords(j)
            pltpu.async_copy(gbufs[b], out_hbm.at[s, bg], wsems[b])

        def write_wait(j, b):
            s, bg = out_coords(j)
            pltpu.make_async_copy(gbufs[b], out_hbm.at[s, bg], wsems[b]).wait()

        prep_idx(0, 0)
        gather_start(0, 0)
        prep_idx(1, 1)
        gather_start(1, 1)
        for j in range(2):
            prep_idx(j + 2, (j + 2) % NBUF)
            gather_start(j + 2, (j + 2) % NBUF)
            gather_wait(j, j % NBUF)
            write_start(j, j % NBUF)

        def steady(ci, carry):
            for u in range(NBUF):
                j = 2 + ci * NBUF + u
                b = (2 + u) % NBUF
                write_wait(j - 2, u % NBUF)
                prep_idx(j + 2, u % NBUF)
                gather_start(j + 2, u % NBUF)
                gather_wait(j, b)
                write_start(j, b)
            return carry

        lax.fori_loop(0, (n_chunks - NBUF) // NBUF, steady, 0)

        for j in range(n_chunks - 2, n_chunks):
            gather_wait(j, j % NBUF)
            write_start(j, j % NBUF)
        for j in range(n_chunks - NBUF, n_chunks):
            write_wait(j, j % NBUF)

    inter = gather_core(table, xt)
    out = _to_native(inter)
    return out.transpose(2, 4, 0, 1, 3).reshape(b_total, seq, D_MODEL)


# speedup vs baseline: 23.5794x; 1.0129x over previous
import functools
import jax
import jax.numpy as jnp
from jax import lax
from jax.experimental import pallas as pl
from jax.experimental.pallas import tpu as pltpu
from jax.experimental.pallas import tpu_sc as plsc

D_MODEL = 64
SCALE = 8.0
LANES = 16
CHUNK = 128
NBUF = 4
UNIT = 8
TBLK = 16384
SB = 4


def _pack_table(lut_t):
    n = lut_t.shape[1]
    grid = (n + TBLK - 1) // TBLK

    def body(in_ref, out_ref):
        x = in_ref[...]
        eye = jnp.eye(D_MODEL, dtype=jnp.float32)
        xt = lax.dot_general(
            x, eye, (((0,), (0,)), ((), ())), preferred_element_type=jnp.float32
        )
        out_ref[...] = jnp.concatenate(
            [xt, jnp.zeros((TBLK, D_MODEL), jnp.float32)], axis=1
        )

    return pl.pallas_call(
        body,
        grid=(grid,),
        in_specs=[pl.BlockSpec((D_MODEL, TBLK), lambda g: (0, g))],
        out_specs=pl.BlockSpec((TBLK, 2 * D_MODEL), lambda g: (g, 0)),
        out_shape=jax.ShapeDtypeStruct((n, 2 * D_MODEL), jnp.float32),
    )(lut_t)


def _to_native(inter):
    seq, nbg = inter.shape[0], inter.shape[1]

    def body(in_ref, out_ref):
        x = in_ref[...]
        x = x.reshape(SB * nbg, CHUNK, CHUNK)
        eye = jnp.eye(CHUNK, dtype=jnp.float32)
        y = lax.dot_general(
            x, eye, (((1,), (0,)), ((), ())), preferred_element_type=jnp.float32
        )
        y = y[:, 0:D_MODEL, :] * SCALE
        y = y.reshape(SB, nbg, D_MODEL // 8, 8, CHUNK).transpose(0, 2, 1, 3, 4)
        out_ref[...] = y

    return pl.pallas_call(
        body,
        grid=(seq // SB,),
        in_specs=[pl.BlockSpec((SB, nbg, CHUNK, CHUNK), lambda s: (s, 0, 0, 0))],
        out_specs=pl.BlockSpec(
            (SB, D_MODEL // 8, nbg, 8, CHUNK), lambda s: (s, 0, 0, 0, 0)
        ),
        out_shape=jax.ShapeDtypeStruct(
            (seq, D_MODEL // 8, nbg, 8, CHUNK), jnp.float32
        ),
    )(inter)


def kernel(lut, x):
    b_total, seq = x.shape
    xt = x.T.astype(jnp.int32).reshape(seq, b_total // 1024, UNIT, CHUNK)
    table = _pack_table(lut.T)

    info = plsc.get_sparse_core_info()
    num_workers = info.num_cores * info.num_subcores
    num_cores = info.num_cores
    chunks_per_s = b_total // CHUNK
    n_chunks = (b_total * seq) // (CHUNK * num_workers)

    mesh = plsc.VectorSubcoreMesh(core_axis_name="c", subcore_axis_name="s")

    @functools.partial(
        pl.kernel,
        mesh=mesh,
        out_type=jax.ShapeDtypeStruct(
            (seq, b_total // CHUNK, CHUNK, CHUNK), jnp.float32
        ),
        scratch_types=[
            pltpu.VMEM((UNIT, CHUNK), jnp.int32),
            [pltpu.VMEM((CHUNK,), jnp.int32) for _ in range(NBUF)],
            [pltpu.VMEM((CHUNK, CHUNK), jnp.float32) for _ in range(NBUF)],
            [pltpu.SemaphoreType.DMA for _ in range(NBUF)],
            [pltpu.SemaphoreType.DMA for _ in range(NBUF)],
        ],
        compiler_params=pltpu.CompilerParams(
            needs_layout_passes=False, use_tc_tiling_on_sc=False
        ),
    )
    def gather_core(lut_hbm, x_hbm, out_hbm, islot, ichunks, gbufs, gsems, wsems):
        wid = lax.axis_index("s") * num_cores + lax.axis_index("c")
        k0 = wid * n_chunks

        def out_coords(j):
            kg = k0 + j
            return kg >> 5, kg & (chunks_per_s - 1)

        def prep_idx(j, b):
            kg = k0 + j
            cc = kg & (UNIT - 1)

            @pl.when(cc == 0)
            def _():
                pltpu.sync_copy(x_hbm.at[kg >> 5, (kg >> 3) & 3], islot)

            for l in range(CHUNK // LANES):
                sl = pl.ds(l * LANES, LANES)
                ichunks[b][sl] = islot[cc, sl]

        def gather_start(j, b):
            pltpu.async_copy(lut_hbm.at[ichunks[b]], gbufs[b], gsems[b])

        def gather_wait(j, b):
            pltpu.make_async_copy(lut_hbm.at[ichunks[b]], gbufs[b], gsems[b]).wait()

        def write_start(j, b):
            s, bg = out_coords(j)
            pltpu.async_copy(gbufs[b], out_hbm.at[s, bg], wsems[b])

        def write_wait(j, b):
            s, bg = out_coords(j)
            pltpu.make_async_copy(gbufs[b], out_hbm.at[s, bg], wsems[b]).wait()

        prep_idx(0, 0)
        gather_start(0, 0)
        prep_idx(1, 1)
        gather_start(1, 1)
        for j in range(2):
            prep_idx(j + 2, (j + 2) % NBUF)
            gather_start(j + 2, (j + 2) % NBUF)
            gather_wait(j, j % NBUF)
            write_start(j, j % NBUF)

        def steady(ci, carry):
            for u in range(NBUF):
                j = 2 + ci * NBUF + u
                b = (2 + u) % NBUF
                write_wait(j - 2, u % NBUF)
                prep_idx(j + 2, u % NBUF)
                gather_start(j + 2, u % NBUF)
                gather_wait(j, b)
                write_start(j, b)
            return carry

        lax.fori_loop(0, (n_chunks - NBUF) // NBUF, steady, 0)

        for j in range(n_chunks - 2, n_chunks):
            gather_wait(j, j % NBUF)
            write_start(j, j % NBUF)
        for j in range(n_chunks - NBUF, n_chunks):
            write_wait(j, j % NBUF)

    inter = gather_core(table, xt)
    out = _to_native(inter)
    return out.transpose(2, 4, 0, 1, 3).reshape(b_total, seq, D_MODEL)


# speedup vs baseline: 23.9556x; 1.0160x over previous
import functools
import jax
import jax.numpy as jnp
from jax import lax
from jax.experimental import pallas as pl
from jax.experimental.pallas import tpu as pltpu
from jax.experimental.pallas import tpu_sc as plsc

D_MODEL = 64
SCALE = 8.0
LANES = 16
CHUNK = 128
NBUF = 4
UNIT = 8
TBLK = 32768
SB = 8


def _pack_table(lut_t):
    n = lut_t.shape[1]
    grid = (n + TBLK - 1) // TBLK

    def body(in_ref, out_ref):
        x = in_ref[...]
        eye = jnp.eye(D_MODEL, dtype=jnp.float32)
        xt = lax.dot_general(
            x, eye, (((0,), (0,)), ((), ())), preferred_element_type=jnp.float32
        )
        out_ref[...] = jnp.concatenate(
            [xt, jnp.zeros((TBLK, D_MODEL), jnp.float32)], axis=1
        )

    return pl.pallas_call(
        body,
        grid=(grid,),
        in_specs=[pl.BlockSpec((D_MODEL, TBLK), lambda g: (0, g))],
        out_specs=pl.BlockSpec((TBLK, 2 * D_MODEL), lambda g: (g, 0)),
        out_shape=jax.ShapeDtypeStruct((n, 2 * D_MODEL), jnp.float32),
    )(lut_t)


def _to_native(inter):
    seq, nbg = inter.shape[0], inter.shape[1]

    def body(in_ref, out_ref):
        x = in_ref[...]
        x = x.reshape(SB * nbg, CHUNK, CHUNK)
        eye = jnp.eye(CHUNK, dtype=jnp.float32)
        y = lax.dot_general(
            x, eye, (((1,), (0,)), ((), ())), preferred_element_type=jnp.float32
        )
        y = y[:, 0:D_MODEL, :] * SCALE
        y = y.reshape(SB, nbg, D_MODEL // 8, 8, CHUNK).transpose(0, 2, 1, 3, 4)
        out_ref[...] = y

    return pl.pallas_call(
        body,
        grid=(seq // SB,),
        in_specs=[pl.BlockSpec((SB, nbg, CHUNK, CHUNK), lambda s: (s, 0, 0, 0))],
        out_specs=pl.BlockSpec(
            (SB, D_MODEL // 8, nbg, 8, CHUNK), lambda s: (s, 0, 0, 0, 0)
        ),
        out_shape=jax.ShapeDtypeStruct(
            (seq, D_MODEL // 8, nbg, 8, CHUNK), jnp.float32
        ),
    )(inter)


def kernel(lut, x):
    b_total, seq = x.shape
    xt = x.T.astype(jnp.int32).reshape(seq, b_total // 1024, UNIT, CHUNK)
    table = _pack_table(lut.T)

    info = plsc.get_sparse_core_info()
    num_workers = info.num_cores * info.num_subcores
    num_cores = info.num_cores
    chunks_per_s = b_total // CHUNK
    n_chunks = (b_total * seq) // (CHUNK * num_workers)

    mesh = plsc.VectorSubcoreMesh(core_axis_name="c", subcore_axis_name="s")

    @functools.partial(
        pl.kernel,
        mesh=mesh,
        out_type=jax.ShapeDtypeStruct(
            (seq, b_total // CHUNK, CHUNK, CHUNK), jnp.float32
        ),
        scratch_types=[
            pltpu.VMEM((UNIT, CHUNK), jnp.int32),
            [pltpu.VMEM((CHUNK,), jnp.int32) for _ in range(NBUF)],
            [pltpu.VMEM((CHUNK, CHUNK), jnp.float32) for _ in range(NBUF)],
            [pltpu.SemaphoreType.DMA for _ in range(NBUF)],
            [pltpu.SemaphoreType.DMA for _ in range(NBUF)],
        ],
        compiler_params=pltpu.CompilerParams(
            needs_layout_passes=False, use_tc_tiling_on_sc=False
        ),
    )
    def gather_core(lut_hbm, x_hbm, out_hbm, islot, ichunks, gbufs, gsems, wsems):
        wid = lax.axis_index("s") * num_cores + lax.axis_index("c")
        k0 = wid * n_chunks

        def out_coords(j):
            kg = k0 + j
            return kg >> 5, kg & (chunks_per_s - 1)

        def prep_idx(j, b):
            kg = k0 + j
            cc = kg & (UNIT - 1)

            @pl.when(cc == 0)
            def _():
                pltpu.sync_copy(x_hbm.at[kg >> 5, (kg >> 3) & 3], islot)

            for l in range(CHUNK // LANES):
                sl = pl.ds(l * LANES, LANES)
                ichunks[b][sl] = islot[cc, sl]

        def gather_start(j, b):
            pltpu.async_copy(lut_hbm.at[ichunks[b]], gbufs[b], gsems[b])

        def gather_wait(j, b):
            pltpu.make_async_copy(lut_hbm.at[ichunks[b]], gbufs[b], gsems[b]).wait()

        def write_start(j, b):
            s, bg = out_coords(j)
            pltpu.async_copy(gbufs[b], out_hbm.at[s, bg], wsems[b])

        def write_wait(j, b):
            s, bg = out_coords(j)
            pltpu.make_async_copy(gbufs[b], out_hbm.at[s, bg], wsems[b]).wait()

        prep_idx(0, 0)
        gather_start(0, 0)
        prep_idx(1, 1)
        gather_start(1, 1)
        for j in range(2):
            prep_idx(j + 2, (j + 2) % NBUF)
            gather_start(j + 2, (j + 2) % NBUF)
            gather_wait(j, j % NBUF)
            write_start(j, j % NBUF)

        def steady(ci, carry):
            for u in range(NBUF):
                j = 2 + ci * NBUF + u
                b = (2 + u) % NBUF
                write_wait(j - 2, u % NBUF)
                prep_idx(j + 2, u % NBUF)
                gather_start(j + 2, u % NBUF)
                gather_wait(j, b)
                write_start(j, b)
            return carry

        lax.fori_loop(0, (n_chunks - NBUF) // NBUF, steady, 0)

        for j in range(n_chunks - 2, n_chunks):
            gather_wait(j, j % NBUF)
            write_start(j, j % NBUF)
        for j in range(n_chunks - NBUF, n_chunks):
            write_wait(j, j % NBUF)

    inter = gather_core(table, xt)
    out = _to_native(inter)
    return out.transpose(2, 4, 0, 1, 3).reshape(b_total, seq, D_MODEL)
